# Initial kernel scaffold; baseline (speedup 1.0000x reference)
import jax
import jax.numpy as jnp
from jax.experimental import pallas as pl


def kernel(point_cloud, cls_label, params):
    raise NotImplementedError("write your pallas kernel here")


# speedup vs baseline: 4.9461x; 4.9461x over previous
import functools

import jax
import jax.numpy as jnp
import numpy as np
from jax.experimental import pallas as pl

_BN_EPS = 1e-5


def _fold(layers):
    out = []
    s = 1.0 / np.sqrt(1.0 + _BN_EPS)
    for (W, b, g, be) in layers:
        sc = g * s
        out.append((W * sc[None, :], (b * sc + be).reshape(1, -1)))
    return out


def _sqdist(src, dst):
    d = -2.0 * jnp.matmul(src, dst.transpose(0, 2, 1))
    d = d + jnp.sum(src ** 2, -1)[:, :, None]
    d = d + jnp.sum(dst ** 2, -1)[:, None, :]
    return d


def _fps(xyz, npoint):
    B, N, _ = xyz.shape

    def body(i, state):
        centroids, distance, farthest = state
        centroids = centroids.at[:, i].set(farthest)
        centroid = jnp.take_along_axis(xyz, farthest[:, None, None], axis=1)
        dist = jnp.sum((xyz - centroid) ** 2, -1)
        distance = jnp.minimum(distance, dist)
        farthest = jnp.argmax(distance, axis=-1).astype(jnp.int32)
        return (centroids, distance, farthest)

    init = (jnp.zeros((B, npoint), jnp.int32),
            jnp.full((B, N), 1e10, jnp.float32),
            jnp.zeros((B,), jnp.int32))
    centroids, _, _ = jax.lax.fori_loop(0, npoint, body, init)
    return centroids


def _query_ball(radius, nsample, xyz, new_xyz):
    B, N, _ = xyz.shape
    S = new_xyz.shape[1]
    sqr = _sqdist(new_xyz, xyz)
    idx = jnp.broadcast_to(jnp.arange(N, dtype=jnp.int32)[None, None, :], (B, S, N))
    idx = jnp.where(sqr > radius ** 2, N, idx)
    idx = jnp.sort(idx, axis=-1)[:, :, :nsample]
    first = jnp.broadcast_to(idx[:, :, :1], idx.shape)
    idx = jnp.where(idx == N, first, idx)
    return idx


def _sa_body(ns, cpc, nlayers, tab_ref, gidx_ref, cpad_ref, *refs):
    out_ref = refs[-1]
    tab = tab_ref[0]
    gi = gidx_ref[0]
    cpad = cpad_ref[0]
    N = tab.shape[0]
    Lc = gi.shape[0]
    oh = (jax.lax.broadcasted_iota(jnp.int32, (Lc, N), 1) == gi).astype(jnp.float32)
    grouped = jnp.dot(oh, tab, preferred_element_type=jnp.float32)
    rsel = (jax.lax.broadcasted_iota(jnp.int32, (Lc, cpc), 0) // ns
            == jax.lax.broadcasted_iota(jnp.int32, (Lc, cpc), 1)).astype(jnp.float32)
    grouped = grouped - jnp.dot(rsel, cpad, preferred_element_type=jnp.float32)
    x = grouped
    for i in range(nlayers):
        W = refs[2 * i][...]
        b = refs[2 * i + 1][...]
        x = jnp.maximum(jnp.dot(x, W, preferred_element_type=jnp.float32) + b, 0.0)
    x = x.reshape(cpc, ns, x.shape[-1])
    out_ref[0] = jnp.max(x, axis=1)


def _ga_body(nlayers, g_ref, *refs):
    out_ref = refs[-1]
    x = g_ref[0]
    for i in range(nlayers):
        W = refs[2 * i][...]
        b = refs[2 * i + 1][...]
        x = jnp.maximum(jnp.dot(x, W, preferred_element_type=jnp.float32) + b, 0.0)
    out_ref[0] = jnp.max(x, axis=0, keepdims=True)


def _fp1_body(p1_ref, p2_ref, wa_ref, wb_ref, b1_ref, w2_ref, b2_ref, out_ref):
    p1 = p1_ref[0]
    p2 = p2_ref[0]
    h = jnp.maximum(
        jnp.dot(p1, wa_ref[...], preferred_element_type=jnp.float32)
        + jnp.dot(p2, wb_ref[...], preferred_element_type=jnp.float32)
        + b1_ref[...], 0.0)
    out_ref[0] = jnp.maximum(
        jnp.dot(h, w2_ref[...], preferred_element_type=jnp.float32) + b2_ref[...], 0.0)


def _fp_body(has_head, x1_ref, x2_ref, p1_ref, p2_ref, *refs):
    out_ref = refs[-1]
    x1 = x1_ref[0]
    x2 = x2_ref[0]
    p1 = p1_ref[0]
    p2 = p2_ref[0]
    N1 = x1.shape[0]
    S2 = x2.shape[0]
    d = (-2.0 * jax.lax.dot_general(x1, x2, (((1,), (1,)), ((), ())),
                                    preferred_element_type=jnp.float32)
         + jnp.sum(x1 * x1, axis=1, keepdims=True)
         + jnp.sum(x2 * x2, axis=1, keepdims=True).reshape(1, S2))
    dd = d
    Wm = jnp.zeros((N1, S2), jnp.float32)
    wsum = jnp.zeros((N1, 1), jnp.float32)
    lane = jax.lax.broadcasted_iota(jnp.int32, (N1, S2), 1)
    for _ in range(3):
        mn = jnp.min(dd, axis=1, keepdims=True)
        am = jnp.argmin(dd, axis=1).astype(jnp.int32)
        oh = (lane == am[:, None]).astype(jnp.float32)
        w = 1.0 / (mn + 1e-8)
        Wm = Wm + oh * w
        wsum = wsum + w
        dd = jnp.where(oh > 0.0, jnp.float32(1e30), dd)
    Wm = Wm / wsum
    interp = jnp.dot(Wm, p2, preferred_element_type=jnp.float32)
    h = jnp.maximum(
        jnp.dot(p1, refs[0][...], preferred_element_type=jnp.float32)
        + jnp.dot(interp, refs[1][...], preferred_element_type=jnp.float32)
        + refs[2][...], 0.0)
    h = jnp.maximum(jnp.dot(h, refs[3][...], preferred_element_type=jnp.float32)
                    + refs[4][...], 0.0)
    if has_head:
        h = jnp.maximum(jnp.dot(h, refs[5][...], preferred_element_type=jnp.float32)
                        + refs[6][...], 0.0)
        lg = jnp.dot(h, refs[7][...], preferred_element_type=jnp.float32) + refs[8][...]
        mx = jnp.max(lg, axis=1, keepdims=True)
        lse = jnp.log(jnp.sum(jnp.exp(lg - mx), axis=1, keepdims=True))
        out_ref[0] = lg - mx - lse
    else:
        out_ref[0] = h


def _wspecs(pairs, ngrid):
    specs, args = [], []
    zmap2 = (lambda b, c: (0, 0)) if ngrid == 2 else (lambda b: (0, 0))
    for (W, b) in pairs:
        specs.append(pl.BlockSpec(W.shape, zmap2))
        specs.append(pl.BlockSpec(b.shape, zmap2))
        args.append(W)
        args.append(b)
    return specs, args


def _sa_call(tab, gidx, cpad, layers, ns):
    B, N, C3 = tab.shape
    S = cpad.shape[1]
    cpc = max(1, 2048 // ns)
    Lc = cpc * ns
    chunks = S // cpc
    gflat = gidx.reshape(B, S * ns, 1).astype(jnp.int32)
    Cout = layers[-1][0].shape[1]
    in_specs = [
        pl.BlockSpec((1, N, C3), lambda b, c: (b, 0, 0)),
        pl.BlockSpec((1, Lc, 1), lambda b, c: (b, c, 0)),
        pl.BlockSpec((1, cpc, C3), lambda b, c: (b, c, 0)),
    ]
    wsp, wargs = _wspecs(layers, 2)
    return pl.pallas_call(
        functools.partial(_sa_body, ns, cpc, len(layers)),
        grid=(B, chunks),
        in_specs=in_specs + wsp,
        out_specs=pl.BlockSpec((1, cpc, Cout), lambda b, c: (b, c, 0)),
        out_shape=jax.ShapeDtypeStruct((B, S, Cout), jnp.float32),
    )(tab, gflat, cpad, *wargs)


def _ga_call(grouped, layers):
    B, S, C = grouped.shape
    Cout = layers[-1][0].shape[1]
    wsp, wargs = _wspecs(layers, 1)
    return pl.pallas_call(
        functools.partial(_ga_body, len(layers)),
        grid=(B,),
        in_specs=[pl.BlockSpec((1, S, C), lambda b: (b, 0, 0))] + wsp,
        out_specs=pl.BlockSpec((1, 1, Cout), lambda b: (b, 0, 0)),
        out_shape=jax.ShapeDtypeStruct((B, 1, Cout), jnp.float32),
    )(grouped, *wargs)


def _fp1_call(p1, p2, layers, c1):
    B, S, C1 = p1.shape
    (W1, b1), (W2, b2) = layers
    Wa, Wb = W1[:c1], W1[c1:]
    Cout = W2.shape[1]
    pairs = [(Wa, b1), (Wb, b1), (W2, b2)]
    in_specs = [
        pl.BlockSpec((1, S, C1), lambda b: (b, 0, 0)),
        pl.BlockSpec((1, 1, p2.shape[2]), lambda b: (b, 0, 0)),
        pl.BlockSpec(Wa.shape, lambda b: (0, 0)),
        pl.BlockSpec(Wb.shape, lambda b: (0, 0)),
        pl.BlockSpec(b1.shape, lambda b: (0, 0)),
        pl.BlockSpec(W2.shape, lambda b: (0, 0)),
        pl.BlockSpec(b2.shape, lambda b: (0, 0)),
    ]
    del pairs
    return pl.pallas_call(
        _fp1_body,
        grid=(B,),
        in_specs=in_specs,
        out_specs=pl.BlockSpec((1, S, Cout), lambda b: (b, 0, 0)),
        out_shape=jax.ShapeDtypeStruct((B, S, Cout), jnp.float32),
    )(p1, p2, Wa, Wb, b1, W2, b2)


def _fp_call(x1, x2, p1, p2, layers, c1, head=None):
    B, N1, _ = x1.shape
    S2 = x2.shape[1]
    C1 = p1.shape[2]
    C2 = p2.shape[2]
    (W1, b1), (W2, b2) = layers
    Wa, Wb = W1[:c1], W1[c1:]
    wargs = [Wa, Wb, b1, W2, b2]
    if head is not None:
        wargs += list(head)
        Cout = head[-2].shape[1]
    else:
        Cout = W2.shape[1]
    in_specs = [
        pl.BlockSpec((1, N1, 3), lambda b: (b, 0, 0)),
        pl.BlockSpec((1, S2, 3), lambda b: (b, 0, 0)),
        pl.BlockSpec((1, N1, C1), lambda b: (b, 0, 0)),
        pl.BlockSpec((1, S2, C2), lambda b: (b, 0, 0)),
    ] + [pl.BlockSpec(w.shape, lambda b: (0, 0)) for w in wargs]
    return pl.pallas_call(
        functools.partial(_fp_body, head is not None),
        grid=(B,),
        in_specs=in_specs,
        out_specs=pl.BlockSpec((1, N1, Cout), lambda b: (b, 0, 0)),
        out_shape=jax.ShapeDtypeStruct((B, N1, Cout), jnp.float32),
    )(x1, x2, p1, p2, *wargs)


def kernel(point_cloud, cls_label, params):
    B, _, N = point_cloud.shape
    pts_t = point_cloud.transpose(0, 2, 1)
    xyz_t = pts_t[..., :3]

    fps1 = _fps(xyz_t, 512)
    new_xyz1 = jnp.take_along_axis(xyz_t, fps1[:, :, None], axis=1)
    tab1 = jnp.concatenate([pts_t, xyz_t], axis=-1)
    cpad1 = jnp.concatenate(
        [jnp.zeros((B, 512, 6), jnp.float32), new_xyz1], axis=-1)
    outs = []
    for radius, ns, mlp in zip([0.1, 0.2, 0.4], [32, 64, 128],
                               [_fold(m) for m in params['sa1']]):
        gidx = _query_ball(radius, ns, xyz_t, new_xyz1)
        outs.append(_sa_call(tab1, gidx, cpad1, mlp, ns))
    pts1_t = jnp.concatenate(outs, axis=-1)

    fps2 = _fps(new_xyz1, 128)
    new_xyz2 = jnp.take_along_axis(new_xyz1, fps2[:, :, None], axis=1)
    tab2 = jnp.concatenate([pts1_t, new_xyz1], axis=-1)
    cpad2 = jnp.concatenate(
        [jnp.zeros((B, 128, 320), jnp.float32), new_xyz2], axis=-1)
    outs = []
    for radius, ns, mlp in zip([0.4, 0.8], [64, 128],
                               [_fold(m) for m in params['sa2']]):
        gidx = _query_ball(radius, ns, new_xyz1, new_xyz2)
        outs.append(_sa_call(tab2, gidx, cpad2, mlp, ns))
    pts2_t = jnp.concatenate(outs, axis=-1)

    grouped3 = jnp.concatenate([new_xyz2, pts2_t], axis=-1)
    pts3 = _ga_call(grouped3, _fold(params['sa3']))

    up1 = _fp1_call(pts2_t, pts3, _fold(params['fp1']), 512)

    up2 = _fp_call(new_xyz1, new_xyz2, pts1_t, up1,
                   _fold(params['fp2']), 320)

    label_b = jnp.broadcast_to(cls_label[:, None, :], (B, N, 16))
    co_t = jnp.concatenate([label_b, xyz_t, pts_t], axis=-1)
    Wc1, bc1, gc1, bec1 = params['cls1']
    (Wc1f, bc1f), = _fold([(Wc1, bc1, gc1, bec1)])
    Wc2, bc2 = params['cls2']
    head = (Wc1f, bc1f, Wc2, bc2.reshape(1, -1))
    pred = _fp_call(xyz_t, new_xyz1, co_t, up2,
                    _fold(params['fp3']), 25, head=head)

    return pred, pts3.transpose(0, 2, 1)
